# Initial kernel scaffold; baseline (speedup 1.0000x reference)
import jax
import jax.numpy as jnp
from jax.experimental import pallas as pl


def kernel(message, _input, x, edge_index, W_gat, a_src, a_dst, W_q, W_k, W_v, W_skip, W_fc, b_fc):
    raise NotImplementedError("write your pallas kernel here")


# speedup vs baseline: 130.5475x; 130.5475x over previous
import functools

import jax
import jax.numpy as jnp
from jax import lax
from jax.experimental import pallas as pl
from jax.experimental.pallas import tpu as pltpu
from jax.experimental.pallas import tpu_sc as plsc

N = 50000
E = 800000
F = 4
EMB = 16
HEADS = 4
OUT_DIM = 64
B = 64
HID = 128

NC = 2
NS = 16
NW = NC * NS
EPW = 25600
EPAD = EPW * NW
CHUNK = 64
NCH = EPW // CHUNK
ROWS_PT = 3128
NPAD = ROWS_PT * NS
REGION = 52000
ACC_W = 40
NBLK = 2000
NGRID = N // NBLK
NPADT = 50176
OBLK = 1792
OGRID = NPADT // OBLK


def _wprep_body(wq_ref, wk_ref, wg_ref, asr_ref, adr_ref, msg_ref, wfc_ref,
                bfc_ref, wo_ref, me_ref):
    wq = wq_ref[...]
    wk = wk_ref[...]
    wg = wg_ref[...]
    a_s = asr_ref[...]
    a_d = adr_ref[...]
    cs = []
    us = []
    ud = []
    for h in range(HEADS):
        wq_h = wq[:, h * EMB:(h + 1) * EMB]
        wk_h = wk[:, h * EMB:(h + 1) * EMB]
        wg_h = wg[:, h * EMB:(h + 1) * EMB]
        cs.append(lax.dot_general(wq_h, wk_h, (((1,), (1,)), ((), ())),
                                  preferred_element_type=jnp.float32) * 0.25)
        us.append(lax.dot_general(wg_h, a_s[h:h + 1, :], (((1,), (1,)), ((), ())),
                                  preferred_element_type=jnp.float32))
        ud.append(lax.dot_general(wg_h, a_d[h:h + 1, :], (((1,), (1,)), ((), ())),
                                  preferred_element_type=jnp.float32))
    ccat = jnp.concatenate(cs, axis=1)
    usm = jnp.concatenate(us, axis=1)
    udm = jnp.concatenate(ud, axis=1)
    r = lax.broadcasted_iota(jnp.int32, (F, F), 0)
    c = lax.broadcasted_iota(jnp.int32, (F, F), 1)
    eye = jnp.where(r == c, 1.0, 0.0).astype(jnp.float32)
    top = jnp.concatenate([eye, usm, ccat, udm,
                           jnp.zeros((F, 88 + 12), jnp.float32)], axis=1)
    wo_ref[...] = jnp.concatenate([top, jnp.zeros((F, 128), jnp.float32)],
                                  axis=0)
    me = jnp.dot(msg_ref[...], wfc_ref[...],
                 preferred_element_type=jnp.float32) + bfc_ref[...]
    me_ref[...] = me


def _nprep_body(x_ref, wo_ref, ts_ref, td_ref, bounds_ref, bmax_ref):
    i = pl.program_id(0)
    wo = wo_ref[...]
    pts = wo[0:F, 0:8]
    ptd = wo[0:F, 8:40]
    x = x_ref[...]
    ts = jnp.dot(x, pts, preferred_element_type=jnp.float32)
    ts_ref[...] = ts
    td = jnp.dot(x, ptd, preferred_element_type=jnp.float32)
    td_ref[...] = td
    z = td[:, 0:16]
    hsel = jnp.where(
        lax.broadcasted_iota(jnp.int32, (16, F), 0) // F
        == lax.broadcasted_iota(jnp.int32, (16, F), 1), 1.0, 0.0
    ).astype(jnp.float32)
    nz2 = jnp.dot(z * z, hsel, preferred_element_type=jnp.float32)
    nx2 = jnp.sum(x * x, axis=1, keepdims=True)
    asb = ts[:, F:2 * F]
    adb = td[:, 16:20]
    bz = jnp.max(nz2, axis=0, keepdims=True)
    bx = jnp.max(nx2, axis=0, keepdims=True)
    bs = jnp.max(asb, axis=0, keepdims=True)
    bd = jnp.max(adb, axis=0, keepdims=True)
    blk = jnp.concatenate(
        [bz, jnp.broadcast_to(bx, (1, F)), bs, bd,
         jnp.full((1, 112), -jnp.inf, jnp.float32)], axis=1)
    blk8 = jnp.broadcast_to(blk, (8, 128))

    @pl.when(i == 0)
    def _():
        bmax_ref[...] = blk8

    @pl.when(i > 0)
    def _():
        bmax_ref[...] = jnp.maximum(bmax_ref[...], blk8)

    bm = bmax_ref[...]
    mzx = jnp.sqrt(jnp.maximum(bm[:, 0:4], 0.0)) * \
        jnp.sqrt(jnp.maximum(bm[:, 4:8], 0.0))
    sg = bm[:, 8:12] + bm[:, 12:16]
    mg = jnp.maximum(sg, 0.0) + 0.2 * jnp.minimum(sg, 0.0)
    bounds_ref[...] = jnp.concatenate(
        [mzx, mg, jnp.zeros((8, 120), jnp.float32)], axis=1)


def _edges_body(ts_hbm, td_hbm, src_hbm, dst_hbm, bnd_hbm, out_hbm,
                acc, si, di, tsb, tdb, pay, vb, sem):
    cid = lax.axis_index("c")
    sid = lax.axis_index("s")
    wid = sid * NC + cid

    pltpu.sync_copy(bnd_hbm.at[0], vb)

    pay_z = jnp.zeros((16,), jnp.float32)

    def _zero_pay(r, _):
        pay[r, pl.ds(0, 16)] = pay_z
        pay[r, pl.ds(16, 16)] = pay_z
        pay[r, pl.ds(24, 16)] = pay_z
        return 0
    lax.fori_loop(0, CHUNK, _zero_pay, 0)
    base_row = sid * ROWS_PT

    def _zero_acc(k, _):
        pltpu.sync_copy(pay, acc.at[pl.ds(base_row + k * CHUNK, CHUNK)])
        return 0
    lax.fori_loop(0, ROWS_PT // CHUNK, _zero_acc, 0)
    rem = ROWS_PT % CHUNK
    if rem:
        pltpu.sync_copy(pay.at[pl.ds(0, rem)],
                        acc.at[pl.ds(base_row + (ROWS_PT // CHUNK) * CHUNK,
                                     rem)])
    plsc.subcore_barrier()

    bv = vb[pl.ds(0, 16)]
    mt = [bv[h] for h in range(HEADS)]
    mg = [bv[F + h] for h in range(HEADS)]

    lanes = lax.iota(jnp.int32, 16)
    ebase0 = wid * EPW

    def _chunk(c, _):
        row0 = wid * NCH + c
        pltpu.sync_copy(src_hbm.at[row0], si)
        pltpu.sync_copy(dst_hbm.at[row0], di)
        cps = [pltpu.async_copy(ts_hbm.at[si], tsb, sem),
               pltpu.async_copy(td_hbm.at[di], tdb, sem)]
        for cp in cps:
            cp.wait()

        def _step(i, _):
            eix = i * 16 + lanes

            def col(k):
                return jnp.full((16,), k, jnp.int32)

            xs = [plsc.load_gather(tsb, [eix, col(f)]) for f in range(F)]
            eg = ebase0 + c * CHUNK + i * 16 + lanes
            mk = jnp.where(eg < E, 1.0, 0.0).astype(jnp.float32)
            for h in range(HEADS):
                zh = [plsc.load_gather(tdb, [eix, col(h * F + f)])
                      for f in range(F)]
                st = zh[0] * xs[0]
                for f in range(1, F):
                    st = st + zh[f] * xs[f]
                asv = plsc.load_gather(tsb, [eix, col(F + h)])
                adv = plsc.load_gather(tdb, [eix, col(16 + h)])
                sg = asv + adv
                sg = jnp.maximum(sg, 0.0) + 0.2 * jnp.minimum(sg, 0.0)
                ext = jnp.exp(st - mt[h]) * mk
                exg = jnp.exp(sg - mg[h]) * mk
                for f in range(F):
                    plsc.store_scatter(pay, [eix, col(h * F + f)],
                                       ext * xs[f])
                    plsc.store_scatter(pay, [eix, col(16 + h * F + f)],
                                       exg * xs[f])
                plsc.store_scatter(pay, [eix, col(32 + h)], ext)
                plsc.store_scatter(pay, [eix, col(36 + h)], exg)
            return 0

        lax.fori_loop(0, CHUNK // 16, _step, 0)
        pltpu.sync_copy(pay, acc.at[di], add=True)
        return 0

    lax.fori_loop(0, NCH, _chunk, 0)
    plsc.subcore_barrier()

    out_base = cid * REGION + sid * ROWS_PT

    def _pub(k, _):
        pltpu.sync_copy(acc.at[pl.ds(base_row + k * CHUNK, CHUNK)],
                        out_hbm.at[pl.ds(out_base + k * CHUNK, CHUNK)])
        return 0
    lax.fori_loop(0, ROWS_PT // CHUNK, _pub, 0)
    if rem:
        pltpu.sync_copy(
            acc.at[pl.ds(base_row + (ROWS_PT // CHUNK) * CHUNK, rem)],
            out_hbm.at[pl.ds(out_base + (ROWS_PT // CHUNK) * CHUNK, rem)])


def _final_body(a0_ref, a1_ref, x_ref, me_ref, wv_ref, wg_ref, wsk_ref,
                logits_ref, m_ref, s_ref, macc, sacc):
    i = pl.program_id(0)
    a = a0_ref[...] + a1_ref[...]
    at = a[:, 0:16]
    ag = a[:, 16:32]
    dent = a[:, 32:36]
    deng = a[:, 36:40]
    rows16 = lax.broadcasted_iota(jnp.int32, (16, OUT_DIM), 0)
    cols16 = lax.broadcasted_iota(jnp.int32, (16, OUT_DIM), 1)
    hmask = jnp.where(rows16 // F == cols16 // EMB, 1.0, 0.0).astype(jnp.float32)
    wv4 = jnp.concatenate([wv_ref[...]] * F, axis=0) * hmask
    wg4 = jnp.concatenate([wg_ref[...]] * F, axis=0) * hmask
    r4 = lax.broadcasted_iota(jnp.int32, (F, OUT_DIM), 0)
    c4 = lax.broadcasted_iota(jnp.int32, (F, OUT_DIM), 1)
    eh = jnp.where(r4 == c4 // EMB, 1.0, 0.0).astype(jnp.float32)
    numt = jnp.dot(at, wv4, preferred_element_type=jnp.float32)
    numg = jnp.dot(ag, wg4, preferred_element_type=jnp.float32)
    dtb = jnp.maximum(jnp.dot(dent, eh, preferred_element_type=jnp.float32),
                      1e-38)
    dgb = jnp.maximum(jnp.dot(deng, eh, preferred_element_type=jnp.float32),
                      1e-38)
    h = numt / dtb + numg / dgb + jnp.dot(
        x_ref[...], wsk_ref[...], preferred_element_type=jnp.float32)
    logits = lax.dot_general(h, me_ref[...], (((1,), (1,)), ((), ())),
                             preferred_element_type=jnp.float32)
    logits_ref[...] = logits

    @pl.when(i == 0)
    def _():
        macc[...] = jnp.full((8, B), -1e30, jnp.float32)
        sacc[...] = jnp.zeros((8, B), jnp.float32)

    m_old = macc[0:1, :]
    bm = jnp.max(logits, axis=0, keepdims=True)
    m_new = jnp.maximum(m_old, bm)
    s_old = sacc[0:1, :]
    s_new = s_old * jnp.exp(m_old - m_new) + jnp.sum(
        jnp.exp(logits - m_new), axis=0, keepdims=True)
    macc[...] = jnp.broadcast_to(m_new, (8, B))
    sacc[...] = jnp.broadcast_to(s_new, (8, B))
    m_ref[...] = macc[...]
    s_ref[...] = sacc[...]


def _norm_body(logits_ref, m_ref, s_ref, out_ref):
    lse = m_ref[0:1, :] + jnp.log(s_ref[0:1, :])
    val = logits_ref[...] - lse
    r = lax.broadcasted_iota(jnp.int32, (B, B), 0)
    c = lax.broadcasted_iota(jnp.int32, (B, B), 1)
    eye = jnp.where(r == c, 1.0, 0.0).astype(jnp.float32)
    out_ref[...] = lax.dot_general(eye, val, (((1,), (1,)), ((), ())),
                                   preferred_element_type=jnp.float32)


def kernel(message, _input, x, edge_index, W_gat, a_src, a_dst, W_q, W_k,
           W_v, W_skip, W_fc, b_fc):
    f32 = jnp.float32
    wo, me = pl.pallas_call(
        _wprep_body,
        out_shape=(jax.ShapeDtypeStruct((8, 128), f32),
                   jax.ShapeDtypeStruct((B, OUT_DIM), f32)),
    )(W_q, W_k, W_gat, a_src, a_dst, message, W_fc,
      b_fc.reshape(1, OUT_DIM))

    tsrc, tdst, bounds = pl.pallas_call(
        _nprep_body,
        grid=(NGRID,),
        in_specs=[pl.BlockSpec((NBLK, F), lambda i: (i, 0)),
                  pl.BlockSpec((8, 128), lambda i: (0, 0))],
        out_specs=[pl.BlockSpec((NBLK, 8), lambda i: (i, 0)),
                   pl.BlockSpec((NBLK, 32), lambda i: (i, 0)),
                   pl.BlockSpec((8, 128), lambda i: (0, 0))],
        out_shape=(jax.ShapeDtypeStruct((N, 8), f32),
                   jax.ShapeDtypeStruct((N, 32), f32),
                   jax.ShapeDtypeStruct((8, 128), f32)),
        scratch_shapes=[pltpu.VMEM((8, 128), f32)],
    )(x, wo)

    src = jnp.pad(edge_index[0], (0, EPAD - E)).reshape(EPAD // CHUNK, CHUNK)
    dst = jnp.pad(edge_index[1], (0, EPAD - E)).reshape(EPAD // CHUNK, CHUNK)

    mesh = plsc.VectorSubcoreMesh(core_axis_name="c", subcore_axis_name="s",
                                  num_cores=NC, num_subcores=NS)
    edges = pl.kernel(
        _edges_body,
        out_type=jax.ShapeDtypeStruct((NC * REGION, ACC_W), f32),
        mesh=mesh,
        compiler_params=pltpu.CompilerParams(needs_layout_passes=False,
                                             use_tc_tiling_on_sc=False),
        scratch_types=[
            pltpu.VMEM_SHARED((NPAD, ACC_W), f32),
            pltpu.VMEM((CHUNK,), jnp.int32),
            pltpu.VMEM((CHUNK,), jnp.int32),
            pltpu.VMEM((CHUNK, 8), f32),
            pltpu.VMEM((CHUNK, 32), f32),
            pltpu.VMEM((CHUNK, ACC_W), f32),
            pltpu.VMEM((128,), f32),
            pltpu.SemaphoreType.DMA,
        ],
    )
    apart = edges(tsrc, tdst, src, dst, bounds)

    logits, m, s = pl.pallas_call(
        _final_body,
        grid=(NGRID,),
        in_specs=[pl.BlockSpec((NBLK, ACC_W), lambda i: (i, 0)),
                  pl.BlockSpec((NBLK, ACC_W), lambda i: (i + REGION // NBLK, 0)),
                  pl.BlockSpec((NBLK, F), lambda i: (i, 0)),
                  pl.BlockSpec((B, OUT_DIM), lambda i: (0, 0)),
                  pl.BlockSpec((F, OUT_DIM), lambda i: (0, 0)),
                  pl.BlockSpec((F, OUT_DIM), lambda i: (0, 0)),
                  pl.BlockSpec((F, OUT_DIM), lambda i: (0, 0))],
        out_specs=[pl.BlockSpec((NBLK, B), lambda i: (i, 0)),
                   pl.BlockSpec((8, B), lambda i: (0, 0)),
                   pl.BlockSpec((8, B), lambda i: (0, 0))],
        out_shape=(jax.ShapeDtypeStruct((NPADT, B), f32),
                   jax.ShapeDtypeStruct((8, B), f32),
                   jax.ShapeDtypeStruct((8, B), f32)),
        scratch_shapes=[pltpu.VMEM((8, B), f32), pltpu.VMEM((8, B), f32)],
    )(apart, apart, x, me, W_v, W_gat, W_skip)

    out = pl.pallas_call(
        _norm_body,
        grid=(OGRID,),
        in_specs=[pl.BlockSpec((OBLK, B), lambda i: (i, 0)),
                  pl.BlockSpec((8, B), lambda i: (0, 0)),
                  pl.BlockSpec((8, B), lambda i: (0, 0))],
        out_specs=pl.BlockSpec((B, OBLK), lambda i: (0, i)),
        out_shape=jax.ShapeDtypeStruct((B, NPADT), f32),
    )(logits, m, s)
    return out[:, :N]
